# baseline (speedup 1.0000x reference)
import functools

import jax
import jax.numpy as jnp
from jax.experimental import pallas as pl
from jax.experimental.pallas import tpu as pltpu


def _deg_proj_kernel(g_ref, x_ref, w_ref, norm_ref, s_ref, *, br, half):
    g = g_ref[...].astype(jnp.bfloat16)
    ones = jnp.ones((g.shape[1], 128), dtype=jnp.bfloat16)
    rs = jnp.dot(g, ones, preferred_element_type=jnp.float32)[:, :1]
    nrm = jax.lax.rsqrt(jnp.maximum(rs, 1.0))
    norm_ref[...] = nrm
    x = x_ref[...]
    h1 = jnp.dot(x, w_ref[0], preferred_element_type=jnp.float32,
                 precision=jax.lax.Precision.HIGHEST)
    h2 = jnp.dot(x, w_ref[1], preferred_element_type=jnp.float32,
                 precision=jax.lax.Precision.HIGHEST)
    rows = pl.program_id(0) * br + jax.lax.broadcasted_iota(
        jnp.int32, (br, 1), 0)
    h = jnp.where(rows < half, h1, h2)
    s_ref[...] = (h * nrm).astype(jnp.bfloat16)


def _spmm_kernel(g_ref, s_ref, norm_ref, out_ref):
    g = g_ref[...].astype(jnp.bfloat16)
    acc = jnp.dot(g, s_ref[...], preferred_element_type=jnp.float32)
    out_ref[...] = acc * norm_ref[...]


def kernel(graph, drug_f, disease_f, drug_w, disease_w):
    n = graph.shape[0]
    half = drug_f.shape[0]
    d = drug_f.shape[1]
    br = 400 if n % 400 == 0 else n
    nblk = n // br

    x = jnp.concatenate([drug_f, disease_f], axis=0)
    w = jnp.stack([drug_w, disease_w], axis=0)

    norm, s = pl.pallas_call(
        functools.partial(_deg_proj_kernel, br=br, half=half),
        grid=(nblk,),
        in_specs=[
            pl.BlockSpec((br, n), lambda i: (i, 0)),
            pl.BlockSpec((br, d), lambda i: (i, 0)),
            pl.BlockSpec((2, d, d), lambda i: (0, 0, 0)),
        ],
        out_specs=[
            pl.BlockSpec((br, 1), lambda i: (i, 0)),
            pl.BlockSpec((br, d), lambda i: (i, 0)),
        ],
        out_shape=[
            jax.ShapeDtypeStruct((n, 1), jnp.float32),
            jax.ShapeDtypeStruct((n, d), jnp.bfloat16),
        ],
        compiler_params=pltpu.CompilerParams(
            dimension_semantics=("parallel",)),
    )(graph, x, w)

    out = pl.pallas_call(
        _spmm_kernel,
        grid=(nblk,),
        in_specs=[
            pl.BlockSpec((br, n), lambda i: (i, 0)),
            pl.BlockSpec((n, d), lambda i: (0, 0)),
            pl.BlockSpec((br, 1), lambda i: (i, 0)),
        ],
        out_specs=pl.BlockSpec((br, d), lambda i: (i, 0)),
        out_shape=jax.ShapeDtypeStruct((n, d), jnp.float32),
        compiler_params=pltpu.CompilerParams(
            dimension_semantics=("parallel",)),
    )(graph, s, norm)
    return out


# speedup vs baseline: 1.0457x; 1.0457x over previous
import functools

import jax
import jax.numpy as jnp
from jax.experimental import pallas as pl
from jax.experimental.pallas import tpu as pltpu


def _fused_kernel(g_ref, x_ref, w_ref, out_ref, s_ref, norm_ref,
                  *, br, half, nblk):
    i = pl.program_id(0)
    n = g_ref.shape[1]
    lo = (i % nblk) * br

    @pl.when(i < nblk)
    def _():
        g = g_ref[...].astype(jnp.bfloat16)
        ones = jnp.ones((n, 128), dtype=jnp.bfloat16)
        rs = jnp.dot(g, ones, preferred_element_type=jnp.float32)[:, :1]
        nrm = jax.lax.rsqrt(jnp.maximum(rs, 1.0))
        norm_ref[pl.ds(lo, br), :] = nrm
        x = x_ref[...]
        h1 = jnp.dot(x, w_ref[0], preferred_element_type=jnp.float32,
                     precision=jax.lax.Precision.HIGHEST)
        h2 = jnp.dot(x, w_ref[1], preferred_element_type=jnp.float32,
                     precision=jax.lax.Precision.HIGHEST)
        rows = lo + jax.lax.broadcasted_iota(jnp.int32, (br, 1), 0)
        h = jnp.where(rows < half, h1, h2)
        s_ref[pl.ds(lo, br), :] = (h * nrm).astype(jnp.bfloat16)

    @pl.when(i >= nblk)
    def _():
        g = g_ref[...].astype(jnp.bfloat16)
        acc = jnp.dot(g, s_ref[...], preferred_element_type=jnp.float32)
        out_ref[...] = acc * norm_ref[pl.ds(lo, br), :]


def kernel(graph, drug_f, disease_f, drug_w, disease_w):
    n = graph.shape[0]
    half = drug_f.shape[0]
    d = drug_f.shape[1]
    br = 400 if n % 400 == 0 else n
    nblk = n // br

    x = jnp.concatenate([drug_f, disease_f], axis=0)
    w = jnp.stack([drug_w, disease_w], axis=0)

    out = pl.pallas_call(
        functools.partial(_fused_kernel, br=br, half=half, nblk=nblk),
        grid=(2 * nblk,),
        in_specs=[
            pl.BlockSpec((br, n), lambda i: (i % nblk, 0)),
            pl.BlockSpec((br, d), lambda i: (i % nblk, 0)),
            pl.BlockSpec((2, d, d), lambda i: (0, 0, 0)),
        ],
        out_specs=pl.BlockSpec(
            (br, d), lambda i: (jnp.maximum(i - nblk, 0), 0)),
        out_shape=jax.ShapeDtypeStruct((n, d), jnp.float32),
        scratch_shapes=[
            pltpu.VMEM((n, d), jnp.bfloat16),
            pltpu.VMEM((n, 1), jnp.float32),
        ],
        compiler_params=pltpu.CompilerParams(
            dimension_semantics=("arbitrary",)),
    )(graph, x, w)
    return out
